# baseline (speedup 1.0000x reference)
import functools
import jax
import jax.numpy as jnp
from jax import lax
from jax.experimental import pallas as pl
from jax.experimental.pallas import tpu as pltpu
from jax.experimental.pallas import tpu_sc as plsc

_D = 32
_V = 100000
_B = 16384
_NC = 2
_NS = 16
_NW = _NC * _NS
_BPW = _B // _NW
_BLK = 128
_NBLK = 25
_TAIL = _V % _BLK

_mesh = plsc.VectorSubcoreMesh(core_axis_name="c", subcore_axis_name="s")


@functools.partial(
    pl.kernel,
    mesh=_mesh,
    out_type=jax.ShapeDtypeStruct((_V * _D,), jnp.float32),
    scratch_types=[
        pltpu.VMEM((2, _D, _BLK + 1), jnp.float32),
        pltpu.VMEM((2, _BLK * _D), jnp.float32),
        pltpu.VMEM((_TAIL * _D,), jnp.float32),
        pltpu.SemaphoreType.DMA,
        pltpu.SemaphoreType.DMA,
        pltpu.SemaphoreType.DMA,
        pltpu.SemaphoreType.DMA,
    ],
    compiler_params=pltpu.CompilerParams(needs_layout_passes=False),
)
def _detile(tt_hbm, tail_hbm, s_hbm, v_in, v_out, tail_v, si0, si1, so0, so1):
    wid = lax.axis_index("s") * _NC + lax.axis_index("c")

    def col_start(k):
        blk = jnp.minimum(wid * _NBLK + k, _V // _BLK - 1)
        return pl.multiple_of(blk * _BLK, _BLK)

    in_sems = (si0, si1)
    out_sems = (so0, so1)

    def in_copy(k, slot):
        return pltpu.make_async_copy(
            tt_hbm.at[:, pl.ds(col_start(k), _BLK)],
            v_in.at[slot, :, pl.ds(0, _BLK)],
            in_sems[slot],
        )

    def out_copy(k, slot):
        return pltpu.make_async_copy(
            v_out.at[slot],
            s_hbm.at[pl.ds(col_start(k) * _D, _BLK * _D)],
            out_sems[slot],
        )

    rows_a = lax.iota(jnp.int32, 16)
    rows_b = rows_a + 16

    zf = jnp.zeros((16,), jnp.float32)

    def transpose_block(slot):
        def body(r, c):
            va, vb = c
            off = (r - 1) * _D

            @pl.when(r > 0)
            def _():
                v_out[slot, pl.ds(off, 16)] = va
                v_out[slot, pl.ds(off + 16, 16)] = vb

            col = jnp.zeros((16,), jnp.int32) + r
            va2 = plsc.load_gather(v_in.at[slot], [rows_a, col])
            vb2 = plsc.load_gather(v_in.at[slot], [rows_b, col])
            return (va2, vb2)

        va, vb = plsc.parallel_loop(0, _BLK, unroll=16, carry=(zf, zf))(body)
        v_out[slot, pl.ds((_BLK - 1) * _D, 16)] = va
        v_out[slot, pl.ds((_BLK - 1) * _D + 16, 16)] = vb

    in_copy(0, 0).start()
    for k in range(_NBLK):
        slot = k % 2
        in_copy(k, slot).wait()
        if k + 1 < _NBLK:
            in_copy(k + 1, 1 - slot).start()
        if k >= 2:
            out_copy(k - 2, slot).wait()
        transpose_block(slot)
        out_copy(k, slot).start()
    out_copy(_NBLK - 2, (_NBLK - 2) % 2).wait()
    out_copy(_NBLK - 1, (_NBLK - 1) % 2).wait()

    @pl.when(wid == 0)
    def _():
        pltpu.sync_copy(tail_hbm, tail_v)
        pltpu.sync_copy(tail_v, s_hbm.at[pl.ds((_V - _TAIL) * _D, _TAIL * _D)])


@functools.partial(
    pl.kernel,
    mesh=_mesh,
    out_type=jax.ShapeDtypeStruct((_B, 2, _D), jnp.float32),
    scratch_types=[
        pltpu.VMEM((_BPW,), jnp.int32),
        pltpu.VMEM((_BPW,), jnp.int32),
        pltpu.VMEM((_BPW, _D), jnp.float32),
        pltpu.VMEM((_BPW, _D), jnp.float32),
        pltpu.SemaphoreType.DMA,
        pltpu.SemaphoreType.DMA,
    ],
    compiler_params=pltpu.CompilerParams(use_tc_tiling_on_sc=False),
)
def _gather_rows(table_hbm, idx_hbm, out_hbm, idx0_v, idx1_v, r0_v, r1_v, s0, s1):
    wid = lax.axis_index("s") * _NC + lax.axis_index("c")
    base = wid * _BPW
    pltpu.sync_copy(idx_hbm.at[0, pl.ds(base, _BPW)], idx0_v)
    pltpu.sync_copy(idx_hbm.at[1, pl.ds(base, _BPW)], idx1_v)
    c0 = pltpu.async_copy(table_hbm.at[idx0_v], r0_v, s0)
    c1 = pltpu.async_copy(table_hbm.at[idx1_v], r1_v, s1)
    c0.wait()
    pltpu.sync_copy(r0_v, out_hbm.at[pl.ds(base, _BPW), 0, :])
    c1.wait()
    pltpu.sync_copy(r1_v, out_hbm.at[pl.ds(base, _BPW), 1, :])


def kernel(action_indices, embedding_table):
    tt = embedding_table.T
    tail = embedding_table[_V - _TAIL :, :].reshape(-1)
    s = _detile(tt, tail)
    table_lin = s.reshape(_V, _D)
    idx_t = action_indices.astype(jnp.int32).T
    out3 = _gather_rows(table_lin, idx_t)
    return out3.reshape(_B, 2 * _D)


# speedup vs baseline: 1.4702x; 1.4702x over previous
import functools
import jax
import jax.numpy as jnp
from jax import lax
from jax.experimental import pallas as pl
from jax.experimental.pallas import tpu as pltpu
from jax.experimental.pallas import tpu_sc as plsc

_D = 32
_V = 100000
_B = 16384
_NC = 2
_NS = 16
_NW = _NC * _NS
_BPW = _B // _NW
_BLK = 128
_NBLK = 25
_TAIL = _V % _BLK

_mesh = plsc.VectorSubcoreMesh(core_axis_name="c", subcore_axis_name="s")


@functools.partial(
    pl.kernel,
    mesh=_mesh,
    out_type=jax.ShapeDtypeStruct((_V * _D,), jnp.float32),
    scratch_types=[
        pltpu.VMEM((2, _D, _BLK), jnp.float32),
        pltpu.VMEM((2, _BLK * _D), jnp.float32),
        pltpu.VMEM((_TAIL * _D,), jnp.float32),
        pltpu.SemaphoreType.DMA,
        pltpu.SemaphoreType.DMA,
        pltpu.SemaphoreType.DMA,
        pltpu.SemaphoreType.DMA,
    ],
    compiler_params=pltpu.CompilerParams(needs_layout_passes=False),
)
def _detile(tt_hbm, tail_hbm, s_hbm, v_in, v_out, tail_v, si0, si1, so0, so1):
    wid = lax.axis_index("s") * _NC + lax.axis_index("c")

    def col_start(k):
        blk = jnp.minimum(wid * _NBLK + k, _V // _BLK - 1)
        return pl.multiple_of(blk * _BLK, _BLK)

    in_sems = (si0, si1)
    out_sems = (so0, so1)

    def in_copy(k, slot):
        return pltpu.make_async_copy(
            tt_hbm.at[:, pl.ds(col_start(k), _BLK)], v_in.at[slot], in_sems[slot]
        )

    def out_copy(k, slot):
        return pltpu.make_async_copy(
            v_out.at[slot],
            s_hbm.at[pl.ds(col_start(k) * _D, _BLK * _D)],
            out_sems[slot],
        )

    lanes = lax.iota(jnp.int32, 16)

    def transpose_block(slot):
        slot_v = jnp.zeros((16,), jnp.int32) + slot

        @plsc.parallel_loop(0, _BLK, unroll=8)
        def _(r):
            col = lax.bitwise_and(lanes + r, _BLK - 1)
            col32 = lax.shift_left(col, 5)
            for jblk in range(_D // 16):
                rows = lanes + jblk * 16
                vals = plsc.load_gather(v_in, [slot_v, rows, col])
                plsc.store_scatter(
                    v_out, [slot_v, col32 + (jblk * 16) + lanes], vals
                )

    in_copy(0, 0).start()
    for k in range(_NBLK):
        slot = k % 2
        in_copy(k, slot).wait()
        if k + 1 < _NBLK:
            in_copy(k + 1, 1 - slot).start()
        if k >= 2:
            out_copy(k - 2, slot).wait()
        transpose_block(slot)
        out_copy(k, slot).start()
    out_copy(_NBLK - 2, (_NBLK - 2) % 2).wait()
    out_copy(_NBLK - 1, (_NBLK - 1) % 2).wait()

    @pl.when(wid == 0)
    def _():
        pltpu.sync_copy(tail_hbm, tail_v)
        pltpu.sync_copy(tail_v, s_hbm.at[pl.ds((_V - _TAIL) * _D, _TAIL * _D)])


@functools.partial(
    pl.kernel,
    mesh=_mesh,
    out_type=jax.ShapeDtypeStruct((_B, 2, _D), jnp.float32),
    scratch_types=[
        pltpu.VMEM((_BPW,), jnp.int32),
        pltpu.VMEM((_BPW,), jnp.int32),
        pltpu.VMEM((_BPW, _D), jnp.float32),
        pltpu.VMEM((_BPW, _D), jnp.float32),
        pltpu.SemaphoreType.DMA,
        pltpu.SemaphoreType.DMA,
    ],
    compiler_params=pltpu.CompilerParams(use_tc_tiling_on_sc=False),
)
def _gather_rows(table_hbm, idx_hbm, out_hbm, idx0_v, idx1_v, r0_v, r1_v, s0, s1):
    wid = lax.axis_index("s") * _NC + lax.axis_index("c")
    base = wid * _BPW
    pltpu.sync_copy(idx_hbm.at[0, pl.ds(base, _BPW)], idx0_v)
    pltpu.sync_copy(idx_hbm.at[1, pl.ds(base, _BPW)], idx1_v)
    c0 = pltpu.async_copy(table_hbm.at[idx0_v], r0_v, s0)
    c1 = pltpu.async_copy(table_hbm.at[idx1_v], r1_v, s1)
    c0.wait()
    pltpu.sync_copy(r0_v, out_hbm.at[pl.ds(base, _BPW), 0, :])
    c1.wait()
    pltpu.sync_copy(r1_v, out_hbm.at[pl.ds(base, _BPW), 1, :])


def kernel(action_indices, embedding_table):
    tt = embedding_table.T
    tail = embedding_table[_V - _TAIL :, :].reshape(-1)
    s = _detile(tt, tail)
    table_lin = s.reshape(_V, _D)
    idx_t = action_indices.astype(jnp.int32).T
    out3 = _gather_rows(table_lin, idx_t)
    return out3.reshape(_B, 2 * _D)
